# Initial kernel scaffold; baseline (speedup 1.0000x reference)
import jax
import jax.numpy as jnp
from jax.experimental import pallas as pl


def kernel(x, emb_table):
    raise NotImplementedError("write your pallas kernel here")


# speedup vs baseline: 5.7540x; 5.7540x over previous
import functools

import jax
import jax.numpy as jnp
from jax import lax
from jax.experimental import pallas as pl
from jax.experimental.pallas import tpu as pltpu
from jax.experimental.pallas import tpu_sc as plsc

_NC = 2
_NS = 16
_NW = _NC * _NS
_CHUNK = 128


@functools.lru_cache(maxsize=None)
def _build(n_tokens, vocab, embed):
    per_w = n_tokens // _NW
    n_chunks = per_w // _CHUNK
    assert per_w % _CHUNK == 0

    mesh = plsc.VectorSubcoreMesh(core_axis_name="c", subcore_axis_name="s")

    @functools.partial(
        pl.kernel,
        out_type=jax.ShapeDtypeStruct((n_tokens, embed), jnp.float32),
        mesh=mesh,
        scratch_types=[
            pltpu.VMEM((n_chunks, _CHUNK), jnp.int32),
            pltpu.VMEM((_CHUNK, embed), jnp.float32),
            pltpu.SemaphoreType.DMA,
        ],
    )
    def _emb(idx_hbm, table_hbm, out_hbm, idx_v, rows_v, gsem):
        wid = lax.axis_index("s") * _NC + lax.axis_index("c")
        base = wid * per_w
        pltpu.sync_copy(idx_hbm.at[wid], idx_v)

        @pl.loop(0, n_chunks)
        def _body(j):
            pltpu.async_copy(table_hbm.at[idx_v.at[j]], rows_v, gsem).wait()
            pltpu.sync_copy(rows_v, out_hbm.at[pl.ds(base + j * _CHUNK, _CHUNK)])

    return _emb


def kernel(x, emb_table):
    B, L = x.shape
    V, D = emb_table.shape
    n = B * L
    emb = _build(n, V, D)
    idx = x.reshape(_NW, n // (_NW * _CHUNK), _CHUNK).astype(jnp.int32)
    out = emb(idx, emb_table)
    return out.reshape(B, L, D)


# speedup vs baseline: 6.5909x; 1.1454x over previous
import functools

import jax
import jax.numpy as jnp
from jax import lax
from jax.experimental import pallas as pl
from jax.experimental.pallas import tpu as pltpu
from jax.experimental.pallas import tpu_sc as plsc

_NC = 2
_NS = 16
_NW = _NC * _NS
_CHUNK = 128


@functools.lru_cache(maxsize=None)
def _build(n_tokens, vocab, embed):
    per_w = n_tokens // _NW
    n_chunks = per_w // _CHUNK
    assert per_w % _CHUNK == 0 and n_chunks % 2 == 0

    mesh = plsc.VectorSubcoreMesh(core_axis_name="c", subcore_axis_name="s")

    @functools.partial(
        pl.kernel,
        out_type=jax.ShapeDtypeStruct((n_tokens, embed), jnp.float32),
        mesh=mesh,
        scratch_types=[
            pltpu.VMEM((n_chunks, _CHUNK), jnp.int32),
            pltpu.VMEM((_CHUNK, embed), jnp.float32),
            pltpu.VMEM((_CHUNK, embed), jnp.float32),
            pltpu.SemaphoreType.DMA,
            pltpu.SemaphoreType.DMA,
            pltpu.SemaphoreType.DMA,
        ],
    )
    def _emb(idx_hbm, table_hbm, out_hbm, idx_v, rows0, rows1, gsem, osem0, osem1):
        wid = lax.axis_index("s") * _NC + lax.axis_index("c")
        base = wid * per_w
        pltpu.sync_copy(idx_hbm.at[wid], idx_v)

        def out_at(j):
            return out_hbm.at[pl.ds(base + j * _CHUNK, _CHUNK)]

        @pl.loop(0, n_chunks, step=2)
        def _body(j):
            @pl.when(j >= 2)
            def _():
                pltpu.make_async_copy(rows0, out_at(j - 2), osem0).wait()

            pltpu.async_copy(table_hbm.at[idx_v.at[j]], rows0, gsem).wait()
            pltpu.async_copy(rows0, out_at(j), osem0)

            @pl.when(j >= 2)
            def _():
                pltpu.make_async_copy(rows1, out_at(j - 1), osem1).wait()

            pltpu.async_copy(table_hbm.at[idx_v.at[j + 1]], rows1, gsem).wait()
            pltpu.async_copy(rows1, out_at(j + 1), osem1)

        pltpu.make_async_copy(rows0, out_at(n_chunks - 2), osem0).wait()
        pltpu.make_async_copy(rows1, out_at(n_chunks - 1), osem1).wait()

    return _emb


def kernel(x, emb_table):
    B, L = x.shape
    V, D = emb_table.shape
    n = B * L
    emb = _build(n, V, D)
    idx = x.reshape(_NW, n // (_NW * _CHUNK), _CHUNK).astype(jnp.int32)
    out = emb(idx, emb_table)
    return out.reshape(B, L, D)


# speedup vs baseline: 7.9896x; 1.2122x over previous
import functools

import jax
import jax.numpy as jnp
from jax import lax
from jax.experimental import pallas as pl
from jax.experimental.pallas import tpu as pltpu
from jax.experimental.pallas import tpu_sc as plsc

_NC = 2
_NS = 16
_NW = _NC * _NS
_CHUNK = 80
_NBUF = 4


@functools.lru_cache(maxsize=None)
def _build(n_tokens, vocab, embed):
    per_w = n_tokens // _NW
    n_chunks = per_w // _CHUNK
    assert per_w % _CHUNK == 0 and n_chunks % _NBUF == 0

    mesh = plsc.VectorSubcoreMesh(core_axis_name="c", subcore_axis_name="s")

    @functools.partial(
        pl.kernel,
        out_type=jax.ShapeDtypeStruct((n_tokens, embed), jnp.float32),
        mesh=mesh,
        scratch_types=[
            pltpu.VMEM((n_chunks, _CHUNK), jnp.int32),
            [pltpu.VMEM((_CHUNK, embed), jnp.float32) for _ in range(_NBUF)],
            [pltpu.SemaphoreType.DMA for _ in range(_NBUF)],
            [pltpu.SemaphoreType.DMA for _ in range(_NBUF)],
        ],
    )
    def _emb(idx_hbm, table_hbm, out_hbm, idx_v, rows, gsem, osem):
        wid = lax.axis_index("s") * _NC + lax.axis_index("c")
        base = wid * per_w
        pltpu.sync_copy(idx_hbm.at[wid], idx_v)

        def out_at(j):
            return out_hbm.at[pl.ds(base + j * _CHUNK, _CHUNK)]

        pltpu.async_copy(table_hbm.at[idx_v.at[0]], rows[0], gsem[0])
        pltpu.async_copy(table_hbm.at[idx_v.at[1]], rows[1], gsem[1])

        @pl.loop(0, n_chunks, step=_NBUF)
        def _body(j):
            for b in range(_NBUF):
                jj = j + b
                b2 = (b + 2) % _NBUF
                @pl.when(jj + 2 < n_chunks)
                def _():
                    @pl.when(jj >= 2)
                    def _():
                        pltpu.make_async_copy(rows[b2], out_at(jj - 2), osem[b2]).wait()
                    pltpu.async_copy(
                        table_hbm.at[idx_v.at[jj + 2]], rows[b2], gsem[b2])
                pltpu.make_async_copy(
                    table_hbm.at[idx_v.at[jj]], rows[b], gsem[b]).wait()
                pltpu.async_copy(rows[b], out_at(jj), osem[b])

        for t in range(_NBUF):
            jj = n_chunks - _NBUF + t
            pltpu.make_async_copy(rows[jj % _NBUF], out_at(jj), osem[jj % _NBUF]).wait()

    return _emb


def kernel(x, emb_table):
    B, L = x.shape
    V, D = emb_table.shape
    n = B * L
    emb = _build(n, V, D)
    idx = x.reshape(_NW, n // (_NW * _CHUNK), _CHUNK).astype(jnp.int32)
    out = emb(idx, emb_table)
    return out.reshape(B, L, D)
